# baseline (speedup 1.0000x reference)
import functools

import jax
import jax.numpy as jnp
from jax import lax
from jax.experimental import pallas as pl
from jax.experimental.pallas import tpu as pltpu
from jax.experimental.pallas import tpu_sc as plsc

B = 16384
E = 32
K = 20
W = K + 1
NC, NS, L = 2, 16, 16
NW = NC * NS
BT = B // NW
C = 128
NCHUNK = BT // C
SW = 24
WROWS = (C * W) // C

_sc_mesh = plsc.VectorSubcoreMesh(core_axis_name="c", subcore_axis_name="s")

VOC = 1000000
NBLK = VOC // 128
VTAIL = NBLK * 128
ORPB = 128 * E // 128
OUT_ROWS = VOC * E // 128
PITCH = 133


@functools.partial(
    pl.kernel,
    out_type=(jax.ShapeDtypeStruct((OUT_ROWS, 128), jnp.float32),
              jax.ShapeDtypeStruct((OUT_ROWS, 128), jnp.float32)),
    mesh=_sc_mesh,
    compiler_params=pltpu.CompilerParams(needs_layout_passes=False),
    scratch_types=[
        pltpu.VMEM((2, E, 128), jnp.float32),
        pltpu.VMEM((2, E, 128), jnp.float32),
        pltpu.VMEM((E * PITCH,), jnp.float32),
        pltpu.VMEM((E, 128), jnp.float32),
        pltpu.VMEM((E, 128), jnp.float32),
        pltpu.SemaphoreType.DMA,
        pltpu.SemaphoreType.DMA,
        pltpu.SemaphoreType.DMA,
        pltpu.SemaphoreType.DMA,
    ],
)
def _sc_transpose(embcT_h, emboT_h, tailc_h, tailo_h, outc_h, outo_h,
                  slabc, slabo, pad_v, obufc, obufo,
                  lsemc, lsemo, ssemc, ssemo):
    wid = lax.axis_index("s") * NC + lax.axis_index("c")
    iota = lax.iota(jnp.int32, L)
    el_lo = iota
    el_hi = iota + L

    def issue_load(src_h, slab_ring, p, j, sem):
        v0 = pl.multiple_of(j * 128, 128)
        pltpu.async_copy(src_h.at[:, pl.ds(v0, 128)], slab_ring.at[p], sem)

    def transpose_block(slab_ring, p, obuf):
        for e in range(E):
            for c in range(128 // L):
                pad_v[pl.ds(e * PITCH + c * L, L)] = (
                    slab_ring[p, e, pl.ds(c * L, L)])
        for v in range(128):
            vcol = jnp.full((L,), v, jnp.int32)
            obuf[v // 4, pl.ds((v % 4) * E, L)] = plsc.load_gather(
                pad_v, [el_lo * PITCH + vcol])
            obuf[v // 4, pl.ds((v % 4) * E + L, L)] = plsc.load_gather(
                pad_v, [el_hi * PITCH + vcol])

    def wait_load(slab_ring, p, sem):
        pltpu.make_async_copy(embcT_h.at[:, pl.ds(0, 128)],
                              slab_ring.at[p], sem).wait()

    def wait_store(obuf, dst_h, sem):
        pltpu.make_async_copy(obuf, dst_h.at[pl.ds(0, ORPB)], sem).wait()

    def issue_store(obuf, dst_h, j, sem):
        r0 = pl.multiple_of(j * ORPB, 8)
        pltpu.async_copy(obuf, dst_h.at[pl.ds(r0, ORPB)], sem)

    nb_all = NBLK // NW + 1

    for p in range(2):
        issue_load(embcT_h, slabc, p, p * NW + wid, lsemc)
        issue_load(emboT_h, slabo, p, p * NW + wid, lsemo)

    def blk_body(i, _):
        j = i * NW + wid
        p = i % 2

        @pl.when(j < NBLK)
        def _():
            jn = (i + 2) * NW + wid
            wait_load(slabc, p, lsemc)

            @pl.when(i > 0)
            def _():
                wait_store(obufc, outc_h, ssemc)

            transpose_block(slabc, p, obufc)
            issue_store(obufc, outc_h, j, ssemc)

            wait_load(slabo, p, lsemo)

            @pl.when(i > 0)
            def _():
                wait_store(obufo, outo_h, ssemo)

            transpose_block(slabo, p, obufo)
            issue_store(obufo, outo_h, j, ssemo)

            @pl.when(jn < NBLK)
            def _():
                issue_load(embcT_h, slabc, p, jn, lsemc)
                issue_load(emboT_h, slabo, p, jn, lsemo)

        return 0

    lax.fori_loop(0, nb_all, blk_body, 0)
    wait_store(obufc, outc_h, ssemc)
    wait_store(obufo, outo_h, ssemo)

    @pl.when(wid == 0)
    def _():
        pltpu.async_copy(tailc_h, obufc.at[pl.ds(0, 16)], lsemc).wait()
        pltpu.async_copy(obufc.at[pl.ds(0, 16)],
                         outc_h.at[pl.ds(NBLK * ORPB, 16)], lsemc).wait()
        pltpu.async_copy(tailo_h, obufo.at[pl.ds(0, 16)], lsemo).wait()
        pltpu.async_copy(obufo.at[pl.ds(0, 16)],
                         outo_h.at[pl.ds(NBLK * ORPB, 16)], lsemo).wait()


@functools.partial(
    pl.kernel,
    out_type=jax.ShapeDtypeStruct((NW * NCHUNK, SW, C), jnp.float32),
    mesh=_sc_mesh,
    compiler_params=pltpu.CompilerParams(needs_layout_passes=False,
                                         use_tc_tiling_on_sc=False),
    scratch_types=[
        pltpu.VMEM((C,), jnp.int32),
        pltpu.VMEM((SW, C), jnp.int32),
        pltpu.VMEM((C, E), jnp.float32),
        pltpu.VMEM((C * W, E), jnp.float32),
        pltpu.VMEM((SW, C), jnp.float32),
        pltpu.SemaphoreType.DMA,
    ],
)
def _sc_scores(cidx_h, widx_h, embc_h, embo_h, out_h,
               cidx_v, widx_v, crows, wrows, scores_v, sem):
    wid = lax.axis_index("s") * NC + lax.axis_index("c")
    iota = lax.iota(jnp.int32, L)

    def chunk_body(ci, _):
        chunk = wid * NCHUNK + ci
        base = chunk * C
        pltpu.sync_copy(cidx_h.at[pl.ds(base, C)], cidx_v)
        pltpu.sync_copy(widx_h.at[chunk], widx_v)
        handles = [pltpu.async_copy(embc_h.at[cidx_v], crows, sem)]
        for j in range(W):
            handles.append(
                pltpu.async_copy(embo_h.at[widx_v.at[j]],
                                 wrows.at[pl.ds(j * C, C)], sem))
        for h in handles:
            h.wait()

        def group_body(g, _):
            rows = g * L + iota
            wrow0 = rows * W

            def e_body(e, accs):
                ecol = (iota + e) & (E - 1)
                c_e = plsc.load_gather(crows, [rows, ecol])
                return tuple(
                    acc + c_e * plsc.load_gather(wrows, [wrow0 + k, ecol])
                    for k, acc in enumerate(accs))

            accs = lax.fori_loop(
                0, E, e_body,
                tuple(jnp.zeros((L,), jnp.float32) for _ in range(W)))
            for k in range(W):
                scores_v[k, pl.ds(g * L, L)] = accs[k]
            zero = jnp.zeros((L,), jnp.float32)
            for k in range(W, SW):
                scores_v[k, pl.ds(g * L, L)] = zero
            return 0

        lax.fori_loop(0, C // L, group_body, 0)
        pltpu.sync_copy(scores_v, out_h.at[chunk])
        return 0

    lax.fori_loop(0, NCHUNK, chunk_body, 0)


_TC_ROWS = (NW * NCHUNK * SW)
_TC_BLK = _TC_ROWS // 8


def _tc_loss_body(s_ref, o_ref):
    i = pl.program_id(0)
    x = s_ref[...]
    r = lax.broadcasted_iota(jnp.int32, x.shape, 0) % SW
    pos = r == 0
    neg = (r >= 1) & (r <= K)
    v = jax.nn.log_sigmoid(jnp.where(pos, x, -x))
    v = jnp.where(pos | neg, v, 0.0)
    part = jnp.sum(v) * (-1.0 / B)

    @pl.when(i == 0)
    def _():
        o_ref[...] = jnp.zeros_like(o_ref)

    o_ref[...] = o_ref[...] + jnp.reshape(part, (1, 1))


def kernel(center, outside, negative, emb_center, emb_outside):
    cidx = jnp.reshape(center, (B,))
    merged = jnp.concatenate([jnp.reshape(outside, (B, 1)), negative], axis=1)
    widx = jnp.pad(
        jnp.reshape(merged, (B // C, C * W)),
        ((0, 0), (0, C * (SW - W)))).reshape(B // C, SW, C)
    tailc = jnp.reshape(emb_center[VTAIL:], (16, 128))
    tailo = jnp.reshape(emb_outside[VTAIL:], (16, 128))
    rowc, rowo = _sc_transpose(emb_center.T, emb_outside.T, tailc, tailo)
    scores = _sc_scores(cidx, widx,
                        jnp.reshape(rowc, (VOC, E)),
                        jnp.reshape(rowo, (VOC, E)))
    scores2d = jnp.reshape(scores, (_TC_ROWS, C))
    loss = pl.pallas_call(
        _tc_loss_body,
        grid=(_TC_ROWS // _TC_BLK,),
        in_specs=[pl.BlockSpec((_TC_BLK, C), lambda i: (i, 0))],
        out_specs=pl.BlockSpec((1, 1), lambda i: (0, 0)),
        out_shape=jax.ShapeDtypeStruct((1, 1), jnp.float32),
    )(scores2d)
    return loss[0, 0]


# speedup vs baseline: 1.1898x; 1.1898x over previous
import functools

import jax
import jax.numpy as jnp
from jax import lax
from jax.experimental import pallas as pl
from jax.experimental.pallas import tpu as pltpu
from jax.experimental.pallas import tpu_sc as plsc

B = 16384
E = 32
K = 20
W = K + 1
NC, NS, L = 2, 16, 16
NW = NC * NS
BT = B // NW
C = 128
NCHUNK = BT // C
SW = 24
WROWS = (C * W) // C

_sc_mesh = plsc.VectorSubcoreMesh(core_axis_name="c", subcore_axis_name="s")

VOC = 1000000
NBLK = VOC // 128
VTAIL = NBLK * 128
ORPB = 128 * E // 128
OUT_ROWS = VOC * E // 128
PITCH = 129


@functools.partial(
    pl.kernel,
    out_type=(jax.ShapeDtypeStruct((OUT_ROWS, 128), jnp.float32),
              jax.ShapeDtypeStruct((OUT_ROWS, 128), jnp.float32)),
    mesh=_sc_mesh,
    compiler_params=pltpu.CompilerParams(needs_layout_passes=False),
    scratch_types=[
        pltpu.VMEM((E, PITCH), jnp.float32),
        pltpu.VMEM((E, PITCH), jnp.float32),
        pltpu.VMEM((E, PITCH), jnp.float32),
        pltpu.VMEM((E, PITCH), jnp.float32),
        pltpu.VMEM((E, 128), jnp.float32),
        pltpu.VMEM((E, 128), jnp.float32),
        pltpu.SemaphoreType.DMA,
        pltpu.SemaphoreType.DMA,
        pltpu.SemaphoreType.DMA,
        pltpu.SemaphoreType.DMA,
        pltpu.SemaphoreType.DMA,
        pltpu.SemaphoreType.DMA,
    ],
)
def _sc_transpose(embcT_h, emboT_h, tailc_h, tailo_h, outc_h, outo_h,
                  slabc0, slabc1, slabo0, slabo1, obufc, obufo,
                  lsemc0, lsemc1, lsemo0, lsemo1, ssemc, ssemo):
    wid = lax.axis_index("s") * NC + lax.axis_index("c")
    iota = lax.iota(jnp.int32, L)
    el_lo = iota
    el_hi = iota + L

    def issue_load(src_h, slab, j, sem):
        v0 = pl.multiple_of(j * 128, 128)
        pltpu.async_copy(src_h.at[:, pl.ds(v0, 128)],
                         slab.at[:, pl.ds(0, 128)], sem)

    def transpose_block(slab, obuf):
        for v0 in range(0, 128, 4):
            vals = []
            for v in range(v0, v0 + 4):
                vcol = jnp.full((L,), v, jnp.int32)
                vals.append(plsc.load_gather(slab, [el_lo, vcol]))
                vals.append(plsc.load_gather(slab, [el_hi, vcol]))
            for q, v in enumerate(range(v0, v0 + 4)):
                obuf[v // 4, pl.ds((v % 4) * E, L)] = vals[2 * q]
                obuf[v // 4, pl.ds((v % 4) * E + L, L)] = vals[2 * q + 1]

    def wait_load(slab, sem):
        pltpu.make_async_copy(embcT_h.at[:, pl.ds(0, 128)],
                              slab.at[:, pl.ds(0, 128)], sem).wait()

    def wait_store(obuf, dst_h, sem):
        pltpu.make_async_copy(obuf, dst_h.at[pl.ds(0, ORPB)], sem).wait()

    def issue_store(obuf, dst_h, j, sem):
        r0 = pl.multiple_of(j * ORPB, 8)
        pltpu.async_copy(obuf, dst_h.at[pl.ds(r0, ORPB)], sem)

    nb_all = NBLK // NW + 1
    cbufs = ((slabc0, lsemc0), (slabc1, lsemc1))
    obufs = ((slabo0, lsemo0), (slabo1, lsemo1))

    for sub in range(2):
        issue_load(embcT_h, cbufs[sub][0], sub * NW + wid, cbufs[sub][1])
        issue_load(emboT_h, obufs[sub][0], sub * NW + wid, obufs[sub][1])

    def blk_body(i2, _):
        for sub in range(2):
            i = i2 * 2 + sub
            j = i * NW + wid
            slc, lsc = cbufs[sub]
            slo, lso = obufs[sub]

            @pl.when(j < NBLK)
            def _():
                jn = (i + 2) * NW + wid
                wait_load(slc, lsc)

                @pl.when(i > 0)
                def _():
                    wait_store(obufc, outc_h, ssemc)

                transpose_block(slc, obufc)
                issue_store(obufc, outc_h, j, ssemc)

                wait_load(slo, lso)

                @pl.when(i > 0)
                def _():
                    wait_store(obufo, outo_h, ssemo)

                transpose_block(slo, obufo)
                issue_store(obufo, outo_h, j, ssemo)

                @pl.when(jn < NBLK)
                def _():
                    issue_load(embcT_h, slc, jn, lsc)
                    issue_load(emboT_h, slo, jn, lso)

        return 0

    lax.fori_loop(0, (nb_all + 1) // 2, blk_body, 0)
    wait_store(obufc, outc_h, ssemc)
    wait_store(obufo, outo_h, ssemo)

    @pl.when(wid == 0)
    def _():
        pltpu.async_copy(tailc_h, obufc.at[pl.ds(0, 16)], lsemc0).wait()
        pltpu.async_copy(obufc.at[pl.ds(0, 16)],
                         outc_h.at[pl.ds(NBLK * ORPB, 16)], lsemc0).wait()
        pltpu.async_copy(tailo_h, obufo.at[pl.ds(0, 16)], lsemo0).wait()
        pltpu.async_copy(obufo.at[pl.ds(0, 16)],
                         outo_h.at[pl.ds(NBLK * ORPB, 16)], lsemo0).wait()


@functools.partial(
    pl.kernel,
    out_type=jax.ShapeDtypeStruct((NW * NCHUNK, SW, C), jnp.float32),
    mesh=_sc_mesh,
    compiler_params=pltpu.CompilerParams(needs_layout_passes=False,
                                         use_tc_tiling_on_sc=False),
    scratch_types=[
        pltpu.VMEM((C,), jnp.int32),
        pltpu.VMEM((SW, C), jnp.int32),
        pltpu.VMEM((C, E), jnp.float32),
        pltpu.VMEM((C * W, E), jnp.float32),
        pltpu.VMEM((SW, C), jnp.float32),
        pltpu.SemaphoreType.DMA,
    ],
)
def _sc_scores(cidx_h, widx_h, embc_h, embo_h, out_h,
               cidx_v, widx_v, crows, wrows, scores_v, sem):
    wid = lax.axis_index("s") * NC + lax.axis_index("c")
    iota = lax.iota(jnp.int32, L)

    def chunk_body(ci, _):
        chunk = wid * NCHUNK + ci
        base = chunk * C
        pltpu.sync_copy(cidx_h.at[pl.ds(base, C)], cidx_v)
        pltpu.sync_copy(widx_h.at[chunk], widx_v)
        handles = [pltpu.async_copy(embc_h.at[cidx_v], crows, sem)]
        for j in range(W):
            handles.append(
                pltpu.async_copy(embo_h.at[widx_v.at[j]],
                                 wrows.at[pl.ds(j * C, C)], sem))
        for h in handles:
            h.wait()

        def group_body(g, _):
            rows = g * L + iota
            wrow0 = rows * W

            def e_body(e, accs):
                ecol = (iota + e) & (E - 1)
                c_e = plsc.load_gather(crows, [rows, ecol])
                return tuple(
                    acc + c_e * plsc.load_gather(wrows, [wrow0 + k, ecol])
                    for k, acc in enumerate(accs))

            accs = lax.fori_loop(
                0, E, e_body,
                tuple(jnp.zeros((L,), jnp.float32) for _ in range(W)))
            for k in range(W):
                scores_v[k, pl.ds(g * L, L)] = accs[k]
            zero = jnp.zeros((L,), jnp.float32)
            for k in range(W, SW):
                scores_v[k, pl.ds(g * L, L)] = zero
            return 0

        lax.fori_loop(0, C // L, group_body, 0)
        pltpu.sync_copy(scores_v, out_h.at[chunk])
        return 0

    lax.fori_loop(0, NCHUNK, chunk_body, 0)


_TC_ROWS = (NW * NCHUNK * SW)
_TC_BLK = _TC_ROWS // 8


def _tc_loss_body(s_ref, o_ref):
    i = pl.program_id(0)
    x = s_ref[...]
    r = lax.broadcasted_iota(jnp.int32, x.shape, 0) % SW
    pos = r == 0
    neg = (r >= 1) & (r <= K)
    v = jax.nn.log_sigmoid(jnp.where(pos, x, -x))
    v = jnp.where(pos | neg, v, 0.0)
    part = jnp.sum(v) * (-1.0 / B)

    @pl.when(i == 0)
    def _():
        o_ref[...] = jnp.zeros_like(o_ref)

    o_ref[...] = o_ref[...] + jnp.reshape(part, (1, 1))


def kernel(center, outside, negative, emb_center, emb_outside):
    cidx = jnp.reshape(center, (B,))
    merged = jnp.concatenate([jnp.reshape(outside, (B, 1)), negative], axis=1)
    widx = jnp.pad(
        jnp.reshape(merged, (B // C, C * W)),
        ((0, 0), (0, C * (SW - W)))).reshape(B // C, SW, C)
    tailc = jnp.reshape(emb_center[VTAIL:], (16, 128))
    tailo = jnp.reshape(emb_outside[VTAIL:], (16, 128))
    rowc, rowo = _sc_transpose(emb_center.T, emb_outside.T, tailc, tailo)
    scores = _sc_scores(cidx, widx,
                        jnp.reshape(rowc, (VOC, E)),
                        jnp.reshape(rowo, (VOC, E)))
    scores2d = jnp.reshape(scores, (_TC_ROWS, C))
    loss = pl.pallas_call(
        _tc_loss_body,
        grid=(_TC_ROWS // _TC_BLK,),
        in_specs=[pl.BlockSpec((_TC_BLK, C), lambda i: (i, 0))],
        out_specs=pl.BlockSpec((1, 1), lambda i: (0, 0)),
        out_shape=jax.ShapeDtypeStruct((1, 1), jnp.float32),
    )(scores2d)
    return loss[0, 0]


# speedup vs baseline: 1.3466x; 1.1317x over previous
import functools

import jax
import jax.numpy as jnp
from jax import lax
from jax.experimental import pallas as pl
from jax.experimental.pallas import tpu as pltpu
from jax.experimental.pallas import tpu_sc as plsc

B = 16384
E = 32
K = 20
W = K + 1
NC, NS, L = 2, 16, 16
NW = NC * NS
BT = B // NW
C = 128
NCHUNK = BT // C
SW = 24
WROWS = (C * W) // C

_sc_mesh = plsc.VectorSubcoreMesh(core_axis_name="c", subcore_axis_name="s")

VOC = 1000000
BW_ = 256
NBLK = VOC // BW_
VTAIL = NBLK * BW_
ORPB = BW_ * E // 128
OUT_ROWS = VOC * E // 128
PITCH = BW_ + 1


@functools.partial(
    pl.kernel,
    out_type=(jax.ShapeDtypeStruct((OUT_ROWS, 128), jnp.float32),
              jax.ShapeDtypeStruct((OUT_ROWS, 128), jnp.float32)),
    mesh=_sc_mesh,
    compiler_params=pltpu.CompilerParams(needs_layout_passes=False),
    scratch_types=[
        pltpu.VMEM((2, E, PITCH), jnp.float32),
        pltpu.VMEM((2, E, PITCH), jnp.float32),
        pltpu.VMEM((ORPB, 128), jnp.float32),
        pltpu.VMEM((ORPB, 128), jnp.float32),
        pltpu.SemaphoreType.DMA,
        pltpu.SemaphoreType.DMA,
        pltpu.SemaphoreType.DMA,
        pltpu.SemaphoreType.DMA,
    ],
)
def _sc_transpose(embcT_h, emboT_h, tailc_h, tailo_h, outc_h, outo_h,
                  slabc, slabo, obufc, obufo,
                  lsemc, lsemo, ssemc, ssemo):
    wid = lax.axis_index("s") * NC + lax.axis_index("c")
    iota = lax.iota(jnp.int32, L)
    el_lo = iota
    el_hi = iota + L

    def issue_load(src_h, slab, j, sem):
        v0 = pl.multiple_of(j * BW_, 128)
        pltpu.async_copy(src_h.at[:, pl.ds(v0, BW_)],
                         slab.at[:, pl.ds(0, BW_)], sem)

    def transpose_block(slab, obuf):
        def vv_body(vv, _):
            vals = []
            for q in range(8):
                vcol = jnp.zeros((L,), jnp.int32) + (vv * 8 + q)
                vals.append(plsc.load_gather(slab, [el_lo, vcol]))
                vals.append(plsc.load_gather(slab, [el_hi, vcol]))
            for q in range(8):
                row = vv * 2 + q // 4
                obuf[row, pl.ds((q % 4) * E, L)] = vals[2 * q]
                obuf[row, pl.ds((q % 4) * E + L, L)] = vals[2 * q + 1]
            return 0

        lax.fori_loop(0, BW_ // 8, vv_body, 0)

    def wait_load(slab, sem):
        pltpu.make_async_copy(embcT_h.at[:, pl.ds(0, BW_)],
                              slab.at[:, pl.ds(0, BW_)], sem).wait()

    def wait_store(obuf, dst_h, sem):
        pltpu.make_async_copy(obuf, dst_h.at[pl.ds(0, ORPB)], sem).wait()

    def issue_store(obuf, dst_h, j, sem):
        r0 = pl.multiple_of(j * ORPB, 8)
        pltpu.async_copy(obuf, dst_h.at[pl.ds(r0, ORPB)], sem)

    nb_all = NBLK // NW + 1

    for sub in range(2):
        issue_load(embcT_h, slabc.at[sub], sub * NW + wid, lsemc)
        issue_load(emboT_h, slabo.at[sub], sub * NW + wid, lsemo)

    def blk_body(i, _):
        j = i * NW + wid
        p = i % 2
        slc = slabc.at[p]
        slo = slabo.at[p]

        @pl.when(j < NBLK)
        def _():
            jn = (i + 2) * NW + wid
            wait_load(slc, lsemc)

            @pl.when(i > 0)
            def _():
                wait_store(obufc, outc_h, ssemc)

            transpose_block(slc, obufc)
            issue_store(obufc, outc_h, j, ssemc)

            wait_load(slo, lsemo)

            @pl.when(i > 0)
            def _():
                wait_store(obufo, outo_h, ssemo)

            transpose_block(slo, obufo)
            issue_store(obufo, outo_h, j, ssemo)

            @pl.when(jn < NBLK)
            def _():
                issue_load(embcT_h, slc, jn, lsemc)
                issue_load(emboT_h, slo, jn, lsemo)

        return 0

    lax.fori_loop(0, nb_all, blk_body, 0)
    wait_store(obufc, outc_h, ssemc)
    wait_store(obufo, outo_h, ssemo)

    @pl.when(wid == 0)
    def _():
        pltpu.async_copy(tailc_h, obufc.at[pl.ds(0, 16)], lsemc).wait()
        pltpu.async_copy(obufc.at[pl.ds(0, 16)],
                         outc_h.at[pl.ds(NBLK * ORPB, 16)], lsemc).wait()
        pltpu.async_copy(tailo_h, obufo.at[pl.ds(0, 16)], lsemo).wait()
        pltpu.async_copy(obufo.at[pl.ds(0, 16)],
                         outo_h.at[pl.ds(NBLK * ORPB, 16)], lsemo).wait()


@functools.partial(
    pl.kernel,
    out_type=jax.ShapeDtypeStruct((NW * NCHUNK, SW, C), jnp.float32),
    mesh=_sc_mesh,
    compiler_params=pltpu.CompilerParams(needs_layout_passes=False,
                                         use_tc_tiling_on_sc=False),
    scratch_types=[
        pltpu.VMEM((C,), jnp.int32),
        pltpu.VMEM((SW, C), jnp.int32),
        pltpu.VMEM((C, E), jnp.float32),
        pltpu.VMEM((C * W, E), jnp.float32),
        pltpu.VMEM((SW, C), jnp.float32),
        pltpu.SemaphoreType.DMA,
    ],
)
def _sc_scores(cidx_h, widx_h, embc_h, embo_h, out_h,
               cidx_v, widx_v, crows, wrows, scores_v, sem):
    wid = lax.axis_index("s") * NC + lax.axis_index("c")
    iota = lax.iota(jnp.int32, L)

    def chunk_body(ci, _):
        chunk = wid * NCHUNK + ci
        base = chunk * C
        pltpu.sync_copy(cidx_h.at[pl.ds(base, C)], cidx_v)
        pltpu.sync_copy(widx_h.at[chunk], widx_v)
        handles = [pltpu.async_copy(embc_h.at[cidx_v], crows, sem)]
        for j in range(W):
            handles.append(
                pltpu.async_copy(embo_h.at[widx_v.at[j]],
                                 wrows.at[pl.ds(j * C, C)], sem))
        for h in handles:
            h.wait()

        def group_body(g, _):
            rows = g * L + iota
            wrow0 = rows * W

            def e_body(e, accs):
                ecol = (iota + e) & (E - 1)
                c_e = plsc.load_gather(crows, [rows, ecol])
                return tuple(
                    acc + c_e * plsc.load_gather(wrows, [wrow0 + k, ecol])
                    for k, acc in enumerate(accs))

            accs = lax.fori_loop(
                0, E, e_body,
                tuple(jnp.zeros((L,), jnp.float32) for _ in range(W)))
            for k in range(W):
                scores_v[k, pl.ds(g * L, L)] = accs[k]
            zero = jnp.zeros((L,), jnp.float32)
            for k in range(W, SW):
                scores_v[k, pl.ds(g * L, L)] = zero
            return 0

        lax.fori_loop(0, C // L, group_body, 0)
        pltpu.sync_copy(scores_v, out_h.at[chunk])
        return 0

    lax.fori_loop(0, NCHUNK, chunk_body, 0)


_TC_ROWS = (NW * NCHUNK * SW)
_TC_BLK = _TC_ROWS // 8


def _tc_loss_body(s_ref, o_ref):
    i = pl.program_id(0)
    x = s_ref[...]
    r = lax.broadcasted_iota(jnp.int32, x.shape, 0) % SW
    pos = r == 0
    neg = (r >= 1) & (r <= K)
    v = jax.nn.log_sigmoid(jnp.where(pos, x, -x))
    v = jnp.where(pos | neg, v, 0.0)
    part = jnp.sum(v) * (-1.0 / B)

    @pl.when(i == 0)
    def _():
        o_ref[...] = jnp.zeros_like(o_ref)

    o_ref[...] = o_ref[...] + jnp.reshape(part, (1, 1))


def kernel(center, outside, negative, emb_center, emb_outside):
    cidx = jnp.reshape(center, (B,))
    merged = jnp.concatenate([jnp.reshape(outside, (B, 1)), negative], axis=1)
    widx = jnp.pad(
        jnp.reshape(merged, (B // C, C * W)),
        ((0, 0), (0, C * (SW - W)))).reshape(B // C, SW, C)
    tailc = jnp.reshape(emb_center[VTAIL:], (16, 128))
    tailo = jnp.reshape(emb_outside[VTAIL:], (16, 128))
    rowc, rowo = _sc_transpose(emb_center.T, emb_outside.T, tailc, tailo)
    scores = _sc_scores(cidx, widx,
                        jnp.reshape(rowc, (VOC, E)),
                        jnp.reshape(rowo, (VOC, E)))
    scores2d = jnp.reshape(scores, (_TC_ROWS, C))
    loss = pl.pallas_call(
        _tc_loss_body,
        grid=(_TC_ROWS // _TC_BLK,),
        in_specs=[pl.BlockSpec((_TC_BLK, C), lambda i: (i, 0))],
        out_specs=pl.BlockSpec((1, 1), lambda i: (0, 0)),
        out_shape=jax.ShapeDtypeStruct((1, 1), jnp.float32),
    )(scores2d)
    return loss[0, 0]


# speedup vs baseline: 1.5327x; 1.1382x over previous
import functools

import jax
import jax.numpy as jnp
from jax import lax
from jax.experimental import pallas as pl
from jax.experimental.pallas import tpu as pltpu
from jax.experimental.pallas import tpu_sc as plsc

B = 16384
E = 32
K = 20
W = K + 1
NC, NS, L = 2, 16, 16
NW = NC * NS
BT = B // NW
C = 128
NCHUNK = BT // C
SW = 24

_sc_mesh = plsc.VectorSubcoreMesh(core_axis_name="c", subcore_axis_name="s")


@functools.partial(
    pl.kernel,
    out_type=jax.ShapeDtypeStruct((NW * NCHUNK, SW, C), jnp.float32),
    mesh=_sc_mesh,
    compiler_params=pltpu.CompilerParams(needs_layout_passes=False,
                                         use_tc_tiling_on_sc=False),
    scratch_types=[
        pltpu.VMEM((C,), jnp.int32),
        pltpu.VMEM((SW, C), jnp.int32),
        pltpu.VMEM((C, E), jnp.float32),
        pltpu.VMEM((C * W, E), jnp.float32),
        pltpu.VMEM((SW, C), jnp.float32),
        pltpu.SemaphoreType.DMA,
    ],
)
def _sc_scores(cidx_h, widx_h, embc_h, embo_h, out_h,
               cidx_v, widx_v, crows, wrows, scores_v, sem):
    wid = lax.axis_index("s") * NC + lax.axis_index("c")
    iota = lax.iota(jnp.int32, L)

    def chunk_body(ci, _):
        chunk = wid * NCHUNK + ci
        base = chunk * C
        pltpu.sync_copy(cidx_h.at[pl.ds(base, C)], cidx_v)
        pltpu.sync_copy(widx_h.at[chunk], widx_v)
        handles = [pltpu.async_copy(embc_h.at[cidx_v], crows, sem)]
        for j in range(W):
            handles.append(
                pltpu.async_copy(embo_h.at[widx_v.at[j]],
                                 wrows.at[pl.ds(j * C, C)], sem))
        for h in handles:
            h.wait()

        def group_body(g, _):
            rows = g * L + iota
            wrow0 = rows * W

            def e_body(e, accs):
                ecol = (iota + e) & (E - 1)
                c_e = plsc.load_gather(crows, [rows, ecol])
                return tuple(
                    acc + c_e * plsc.load_gather(wrows, [wrow0 + k, ecol])
                    for k, acc in enumerate(accs))

            accs = lax.fori_loop(
                0, E, e_body,
                tuple(jnp.zeros((L,), jnp.float32) for _ in range(W)))
            for k in range(W):
                scores_v[k, pl.ds(g * L, L)] = accs[k]
            zero = jnp.zeros((L,), jnp.float32)
            for k in range(W, SW):
                scores_v[k, pl.ds(g * L, L)] = zero
            return 0

        lax.fori_loop(0, C // L, group_body, 0)
        pltpu.sync_copy(scores_v, out_h.at[chunk])
        return 0

    lax.fori_loop(0, NCHUNK, chunk_body, 0)


_TC_ROWS = (NW * NCHUNK * SW)
_TC_BLK = _TC_ROWS // 8


def _tc_loss_body(s_ref, o_ref):
    i = pl.program_id(0)
    x = s_ref[...]
    r = lax.broadcasted_iota(jnp.int32, x.shape, 0) % SW
    pos = r == 0
    neg = (r >= 1) & (r <= K)
    v = jax.nn.log_sigmoid(jnp.where(pos, x, -x))
    v = jnp.where(pos | neg, v, 0.0)
    part = jnp.sum(v) * (-1.0 / B)

    @pl.when(i == 0)
    def _():
        o_ref[...] = jnp.zeros_like(o_ref)

    o_ref[...] = o_ref[...] + jnp.reshape(part, (1, 1))


def kernel(center, outside, negative, emb_center, emb_outside):
    cidx = jnp.reshape(center, (B,))
    merged = jnp.concatenate([jnp.reshape(outside, (B, 1)), negative], axis=1)
    widx = jnp.pad(
        jnp.reshape(merged, (B // C, C * W)),
        ((0, 0), (0, C * (SW - W)))).reshape(B // C, SW, C)
    scores = _sc_scores(cidx, widx, emb_center, emb_outside)
    scores2d = jnp.reshape(scores, (_TC_ROWS, C))
    loss = pl.pallas_call(
        _tc_loss_body,
        grid=(_TC_ROWS // _TC_BLK,),
        in_specs=[pl.BlockSpec((_TC_BLK, C), lambda i: (i, 0))],
        out_specs=pl.BlockSpec((1, 1), lambda i: (0, 0)),
        out_shape=jax.ShapeDtypeStruct((1, 1), jnp.float32),
    )(scores2d)
    return loss[0, 0]
